# baseline (speedup 1.0000x reference)
import functools

import jax
import jax.numpy as jnp
from jax import lax
from jax.experimental import pallas as pl
from jax.experimental.pallas import tpu as pltpu
from jax.experimental.pallas import tpu_sc as plsc

RADIUS = 0.1
VISCOSITY = 0.01

NC = 2
NS = 16
NW = NC * NS
L = 16

EDGE_CHUNK = 2000
PART_CHUNK = 2000
GB = 128
NPP = 3200


def _edge_prefix_body(M, Np, vflat, m_pad, rho_pad, nbr_idx, nbr_dist,
                      lx, ly, lz, la, tot,
                      vbuf, mbuf, rbuf, cx, cy, cz, ca,
                      spx, spy, spz, spa,
                      idx_a, dist_a, bxa, bya, bza, baa,
                      idx_b, dist_b, bxb, byb, bzb, bab,
                      pxa, pya, pza, paa, pxb, pyb, pzb, pab,
                      tbuf, sem_a, sem_b, sem_oa, sem_ob):
    bufs_a = (bxa, bya, bza, baa)
    bufs_b = (bxb, byb, bzb, bab)
    prefs_a = (pxa, pya, pza, paa)
    prefs_b = (pxb, pyb, pzb, pab)
    sps = (spx, spy, spz, spa)
    s = lax.axis_index("s")
    w = lax.axis_index("c") * NS + s
    iota = lax.iota(jnp.int32, L)
    epw = M // NW
    n_chunks = epw // EDGE_CHUNK
    inv_r2 = jnp.float32(1.0 / (RADIUS * RADIUS))
    zeros_i = jnp.zeros((L,), jnp.int32)

    p0 = s * NPP
    pltpu.sync_copy(vflat.at[pl.ds(p0 * 3, NPP * 3)], vbuf)
    pltpu.sync_copy(m_pad.at[pl.ds(p0, NPP)], mbuf)
    pltpu.sync_copy(rho_pad.at[pl.ds(p0, NPP)], rbuf)

    def bstep(j, _):
        off = j * L
        pvec3 = (iota + off) * 3
        cx[pl.ds(off, L)] = plsc.load_gather(vbuf, [pvec3])
        cy[pl.ds(off, L)] = plsc.load_gather(vbuf, [pvec3 + 1])
        cz[pl.ds(off, L)] = plsc.load_gather(vbuf, [pvec3 + 2])
        ca[pl.ds(off, L)] = mbuf[pl.ds(off, L)] / rbuf[pl.ds(off, L)]
        return 0

    lax.fori_loop(0, NPP // L, bstep, 0)
    for c, sp in zip((cx, cy, cz, ca), sps):
        pltpu.sync_copy(c, sp.at[pl.ds(p0, NPP)])
    plsc.subcore_barrier()

    def fire(c, idx_v, dist_v, bufs, sem):
        base = w * epw + c * EDGE_CHUNK
        pltpu.sync_copy(nbr_idx.at[pl.ds(base, EDGE_CHUNK)],
                        idx_v.at[pl.ds(0, EDGE_CHUNK)])
        pltpu.sync_copy(nbr_dist.at[pl.ds(base, EDGE_CHUNK)],
                        dist_v.at[pl.ds(0, EDGE_CHUNK)])
        for t in range((idx_v.shape[0] - EDGE_CHUNK) // L):
            idx_v[pl.ds(EDGE_CHUNK + t * L, L)] = zeros_i
        for r in range(idx_v.shape[0] // GB):
            isl = idx_v.at[pl.ds(r * GB, GB)]
            for sp, buf in zip(sps, bufs):
                pltpu.async_copy(sp.at[isl], buf.at[pl.ds(r * GB, GB)], sem)

    def drain(bufs, sem):
        for buf in bufs:
            pltpu.make_async_copy(nbr_dist.at[pl.ds(0, buf.shape[0])], buf,
                                  sem).wait()

    def compute_out(c, dist_v, bufs, prefs, sem_out, carry):
        bx, by, bz, ba = bufs
        for pref in prefs:
            pltpu.make_async_copy(nbr_dist.at[pl.ds(0, EDGE_CHUNK)], pref,
                                  sem_out).wait()

        def step(j, carry):
            c0, c1, c2, c3 = carry
            off = j * L
            d = dist_v[pl.ds(off, L)]
            a = ba[pl.ds(off, L)]
            u = 1.0 - d * inv_r2
            wgt = jnp.clip(u * u * u, 0.0, 1.0) * a
            new_carry = []
            for ch, (x, cprev) in enumerate((
                    (wgt * bx[pl.ds(off, L)], c0),
                    (wgt * by[pl.ds(off, L)], c1),
                    (wgt * bz[pl.ds(off, L)], c2),
                    (wgt, c3))):
                inc = plsc.cumsum(x)
                prefs[ch][pl.ds(off, L)] = cprev + (inc - x)
                new_carry.append(cprev + inc[L - 1])
            return tuple(new_carry)

        carry = lax.fori_loop(0, EDGE_CHUNK // L, step, carry)
        base = w * epw + c * EDGE_CHUNK
        for pref, lex in zip(prefs, (lx, ly, lz, la)):
            pltpu.async_copy(pref, lex.at[pl.ds(base, EDGE_CHUNK)], sem_out)
        return carry

    for prefs, sem_out in ((prefs_a, sem_oa), (prefs_b, sem_ob)):
        for pref in prefs:
            pltpu.async_copy(nbr_dist.at[pl.ds(0, EDGE_CHUNK)], pref, sem_out)

    fire(0, idx_a, dist_a, bufs_a, sem_a)

    def pair(k, carry):
        c0 = 2 * k
        drain(bufs_a, sem_a)
        fire(c0 + 1, idx_b, dist_b, bufs_b, sem_b)
        carry = compute_out(c0, dist_a, bufs_a, prefs_a, sem_oa, carry)
        drain(bufs_b, sem_b)
        fire(c0 + 2, idx_a, dist_a, bufs_a, sem_a)
        carry = compute_out(c0 + 1, dist_b, bufs_b, prefs_b, sem_ob, carry)
        return carry

    z = jnp.float32(0.0)
    carry = lax.fori_loop(0, (n_chunks - 1) // 2, pair, (z, z, z, z))
    drain(bufs_a, sem_a)
    c0, c1, c2, c3 = compute_out(n_chunks - 1, dist_a, bufs_a, prefs_a,
                                 sem_oa, carry)
    for prefs, sem_out in ((prefs_a, sem_oa), (prefs_b, sem_ob)):
        for pref in prefs:
            pltpu.make_async_copy(nbr_dist.at[pl.ds(0, EDGE_CHUNK)], pref,
                                  sem_out).wait()

    tvec = jnp.where(iota == 0, c0,
                     jnp.where(iota == 1, c1,
                               jnp.where(iota == 2, c2,
                                         jnp.where(iota == 3, c3, 0.0))))
    tbuf[...] = tvec
    pltpu.sync_copy(tbuf, tot.at[pl.ds(w * L, L)])


def _combine_body(M, Np, vflat, rs_hbm, lx, ly, lz, la, tot, oflat,
                  rs_v, gx, gy, gz, ga, v_v, o_v, t_v, offb, sem):
    w = lax.axis_index("c") * NS + lax.axis_index("s")
    iota = lax.iota(jnp.int32, L)
    epw = M // NW
    n_chunks = Np // PART_CHUNK
    visc = jnp.float32(VISCOSITY)

    base = lax.min(w, n_chunks - 1) * PART_CHUNK

    pltpu.sync_copy(tot, t_v)
    for ch in range(4):
        lo = plsc.load_gather(t_v, [iota * L + ch])
        hi = plsc.load_gather(t_v, [(iota + L) * L + ch])
        inc_lo = plsc.cumsum(lo)
        inc_hi = plsc.cumsum(hi) + inc_lo[L - 1]
        plsc.store_scatter(offb, [ch * 48 + iota], inc_lo - lo)
        plsc.store_scatter(offb, [ch * 48 + L + iota], inc_hi - hi)
        plsc.store_scatter(offb, [ch * 48 + 32 + iota],
                           jnp.full((L,), inc_hi[L - 1], jnp.float32))

    zi = jnp.zeros((L,), jnp.int32)
    for t in range((rs_v.shape[0] - PART_CHUNK) // L):
        rs_v[pl.ds(PART_CHUNK + t * L, L)] = zi
    pltpu.sync_copy(rs_hbm.at[pl.ds(base, PART_CHUNK + 16)],
                    rs_v.at[pl.ds(0, PART_CHUNK + 16)])
    pltpu.sync_copy(vflat.at[pl.ds(base * 3, PART_CHUNK * 3)], v_v)
    descs = []
    for r in range(rs_v.shape[0] // GB):
        isl = rs_v.at[pl.ds(r * GB, GB)]
        for src, buf in ((lx, gx), (ly, gy), (lz, gz), (la, ga)):
            descs.append(pltpu.async_copy(
                src.at[isl], buf.at[pl.ds(r * GB, GB)], sem))
    for dsc in descs:
        dsc.wait()

    mval = jnp.int32(M)

    def step(j, _):
        off = j * L
        pvec = iota + off
        rs_a = rs_v[pl.ds(off, L)]
        rs_b = plsc.load_gather(rs_v, [pvec + 1])
        ta_ = rs_a // epw
        tb_ = rs_b // epw
        svals = []
        for ch, buf in enumerate((ga, gx, gy, gz)):
            pa_ = buf[pl.ds(off, L)]
            pb_ = plsc.load_gather(buf, [pvec + 1])
            och = (3, 0, 1, 2)[ch]
            pa_ = jnp.where(rs_a == mval, 0.0, pa_)
            pb_ = jnp.where(rs_b == mval, 0.0, pb_)
            oa = plsc.load_gather(offb, [och * 48 + ta_])
            ob = plsc.load_gather(offb, [och * 48 + tb_])
            svals.append((pb_ - pa_) + (ob - oa))
        s0, s1x, s1y, s1z = svals
        pvec3 = pvec * 3
        for ch, s1 in enumerate((s1x, s1y, s1z)):
            vch = plsc.load_gather(v_v, [pvec3 + ch])
            plsc.store_scatter(o_v, [pvec3 + ch],
                               vch - visc * (vch * s0 - s1))
        return 0

    lax.fori_loop(0, PART_CHUNK // L, step, 0)

    @pl.when(w < n_chunks)
    def _():
        pltpu.sync_copy(o_v, oflat.at[pl.ds(base * 3, PART_CHUNK * 3)])


def kernel(velocities, masses, densities, neighbors_index,
           neighbors_row_splits, neighbors_distance):
    Np = velocities.shape[0]
    M = neighbors_index.shape[0]
    f32 = jnp.float32
    npad = NS * NPP - Np

    vflat = velocities.reshape(-1)
    vflat_pad = jnp.concatenate([vflat, jnp.zeros((npad * 3,), f32)])
    m_pad = jnp.concatenate([masses, jnp.ones((npad,), f32)])
    rho_pad = jnp.concatenate([densities, jnp.ones((npad,), f32)])

    mesh = plsc.VectorSubcoreMesh(core_axis_name="c", subcore_axis_name="s")
    lex_t = jax.ShapeDtypeStruct((M + 8,), f32)

    edge_kernel = pl.kernel(
        functools.partial(_edge_prefix_body, M, Np),
        out_type=(lex_t, lex_t, lex_t, lex_t,
                  jax.ShapeDtypeStruct((NW * L,), f32)),
        mesh=mesh,
        compiler_params=pltpu.CompilerParams(needs_layout_passes=False),
        scratch_types=(
            [pltpu.VMEM((NPP * 3,), f32)]
            + [pltpu.VMEM((NPP,), f32) for _ in range(6)]
            + [pltpu.VMEM_SHARED((NS * NPP,), f32) for _ in range(4)]
            + [pltpu.VMEM((2048,), jnp.int32), pltpu.VMEM((2048,), f32)]
            + [pltpu.VMEM((2048,), f32) for _ in range(4)]
            + [pltpu.VMEM((2048,), jnp.int32), pltpu.VMEM((2048,), f32)]
            + [pltpu.VMEM((2048,), f32) for _ in range(4)]
            + [pltpu.VMEM((EDGE_CHUNK,), f32) for _ in range(8)]
            + [pltpu.VMEM((L,), f32),
               pltpu.SemaphoreType.DMA, pltpu.SemaphoreType.DMA,
               pltpu.SemaphoreType.DMA, pltpu.SemaphoreType.DMA]
        ),
    )
    lx, ly, lz, la, tot = edge_kernel(vflat_pad, m_pad, rho_pad,
                                      neighbors_index, neighbors_distance)

    pad = jnp.full((63,), M, dtype=neighbors_row_splits.dtype)
    rs_pad = jnp.concatenate([neighbors_row_splits, pad])

    combine_kernel = pl.kernel(
        functools.partial(_combine_body, M, Np),
        out_type=jax.ShapeDtypeStruct((Np * 3,), f32),
        mesh=mesh,
        compiler_params=pltpu.CompilerParams(needs_layout_passes=False),
        scratch_types=(
            [pltpu.VMEM((2048,), jnp.int32)]
            + [pltpu.VMEM((2048,), f32) for _ in range(4)]
            + [pltpu.VMEM((PART_CHUNK * 3,), f32)]
            + [pltpu.VMEM((PART_CHUNK * 3,), f32)]
            + [pltpu.VMEM((NW * L,), f32), pltpu.VMEM((192,), f32),
               pltpu.SemaphoreType.DMA]
        ),
    )
    oflat = combine_kernel(vflat, rs_pad, lx, ly, lz, la, tot)
    return oflat.reshape(Np, 3)


# speedup vs baseline: 1.0314x; 1.0314x over previous
import functools

import jax
import jax.numpy as jnp
from jax import lax
from jax.experimental import pallas as pl
from jax.experimental.pallas import tpu as pltpu
from jax.experimental.pallas import tpu_sc as plsc

RADIUS = 0.1
VISCOSITY = 0.01

NC = 2
NS = 16
NW = NC * NS
L = 16

EDGE_CHUNK = 2000
PART_CHUNK = 2000
GB = 128
NPP = 3200


def _edge_prefix_body(M, Np, vflat, m_pad, rho_pad, nbr_idx, nbr_dist,
                      lx, ly, lz, la, tot,
                      vbuf, mbuf, rbuf, cx, cy, cz, ca,
                      spx, spy, spz, spa,
                      idx_a, dist_a, bxa, bya, bza, baa,
                      idx_b, dist_b, bxb, byb, bzb, bab,
                      px, py, pz, pa, tbuf, sem_a, sem_b):
    bufs_a = (bxa, bya, bza, baa)
    bufs_b = (bxb, byb, bzb, bab)
    prefs = (px, py, pz, pa)
    sps = (spx, spy, spz, spa)
    s = lax.axis_index("s")
    w = lax.axis_index("c") * NS + s
    iota = lax.iota(jnp.int32, L)
    epw = M // NW
    n_chunks = epw // EDGE_CHUNK
    inv_r2 = jnp.float32(1.0 / (RADIUS * RADIUS))
    zeros_i = jnp.zeros((L,), jnp.int32)

    p0 = s * NPP
    pltpu.sync_copy(vflat.at[pl.ds(p0 * 3, NPP * 3)], vbuf)
    pltpu.sync_copy(m_pad.at[pl.ds(p0, NPP)], mbuf)
    pltpu.sync_copy(rho_pad.at[pl.ds(p0, NPP)], rbuf)

    def bstep(j, _):
        off = j * L
        pvec3 = (iota + off) * 3
        cx[pl.ds(off, L)] = plsc.load_gather(vbuf, [pvec3])
        cy[pl.ds(off, L)] = plsc.load_gather(vbuf, [pvec3 + 1])
        cz[pl.ds(off, L)] = plsc.load_gather(vbuf, [pvec3 + 2])
        ca[pl.ds(off, L)] = mbuf[pl.ds(off, L)] / rbuf[pl.ds(off, L)]
        return 0

    lax.fori_loop(0, NPP // L, bstep, 0)
    for c, sp in zip((cx, cy, cz, ca), sps):
        pltpu.sync_copy(c, sp.at[pl.ds(p0, NPP)])
    plsc.subcore_barrier()

    def fire(c, idx_v, dist_v, bufs, sem):
        base = w * epw + c * EDGE_CHUNK
        pltpu.sync_copy(nbr_idx.at[pl.ds(base, EDGE_CHUNK)],
                        idx_v.at[pl.ds(0, EDGE_CHUNK)])
        pltpu.sync_copy(nbr_dist.at[pl.ds(base, EDGE_CHUNK)],
                        dist_v.at[pl.ds(0, EDGE_CHUNK)])
        for t in range((idx_v.shape[0] - EDGE_CHUNK) // L):
            idx_v[pl.ds(EDGE_CHUNK + t * L, L)] = zeros_i
        for r in range(idx_v.shape[0] // GB):
            isl = idx_v.at[pl.ds(r * GB, GB)]
            for sp, buf in zip(sps, bufs):
                pltpu.async_copy(sp.at[isl], buf.at[pl.ds(r * GB, GB)], sem)

    def drain(bufs, sem):
        for buf in bufs:
            pltpu.make_async_copy(nbr_dist.at[pl.ds(0, buf.shape[0])], buf,
                                  sem).wait()

    def compute_out(c, dist_v, bufs, carry):
        bx, by, bz, ba = bufs

        def step(j, carry):
            c0, c1, c2, c3 = carry
            off = j * L
            d = dist_v[pl.ds(off, L)]
            a = ba[pl.ds(off, L)]
            u = 1.0 - d * inv_r2
            wgt = jnp.clip(u * u * u, 0.0, 1.0) * a
            new_carry = []
            for ch, (x, cprev) in enumerate((
                    (wgt * bx[pl.ds(off, L)], c0),
                    (wgt * by[pl.ds(off, L)], c1),
                    (wgt * bz[pl.ds(off, L)], c2),
                    (wgt, c3))):
                inc = plsc.cumsum(x)
                prefs[ch][pl.ds(off, L)] = cprev + (inc - x)
                new_carry.append(cprev + inc[L - 1])
            return tuple(new_carry)

        carry = lax.fori_loop(0, EDGE_CHUNK // L, step, carry)
        base = w * epw + c * EDGE_CHUNK
        for pref, lex in zip(prefs, (lx, ly, lz, la)):
            pltpu.sync_copy(pref, lex.at[pl.ds(base, EDGE_CHUNK)])
        return carry

    fire(0, idx_a, dist_a, bufs_a, sem_a)

    def pair(k, carry):
        c0 = 2 * k
        drain(bufs_a, sem_a)
        fire(c0 + 1, idx_b, dist_b, bufs_b, sem_b)
        carry = compute_out(c0, dist_a, bufs_a, carry)
        drain(bufs_b, sem_b)
        fire(c0 + 2, idx_a, dist_a, bufs_a, sem_a)
        carry = compute_out(c0 + 1, dist_b, bufs_b, carry)
        return carry

    z = jnp.float32(0.0)
    carry = lax.fori_loop(0, (n_chunks - 1) // 2, pair, (z, z, z, z))
    drain(bufs_a, sem_a)
    c0, c1, c2, c3 = compute_out(n_chunks - 1, dist_a, bufs_a, carry)

    tvec = jnp.where(iota == 0, c0,
                     jnp.where(iota == 1, c1,
                               jnp.where(iota == 2, c2,
                                         jnp.where(iota == 3, c3, 0.0))))
    tbuf[...] = tvec
    pltpu.sync_copy(tbuf, tot.at[pl.ds(w * L, L)])


def _combine_body(M, Np, vflat, rs_hbm, lx, ly, lz, la, tot, oflat,
                  rs_v, gx, gy, gz, ga, v_v, o_v, t_v, offb, sem):
    w = lax.axis_index("c") * NS + lax.axis_index("s")
    iota = lax.iota(jnp.int32, L)
    epw = M // NW
    n_chunks = Np // PART_CHUNK
    visc = jnp.float32(VISCOSITY)

    base = lax.min(w, n_chunks - 1) * PART_CHUNK

    pltpu.sync_copy(tot, t_v)
    for ch in range(4):
        lo = plsc.load_gather(t_v, [iota * L + ch])
        hi = plsc.load_gather(t_v, [(iota + L) * L + ch])
        inc_lo = plsc.cumsum(lo)
        inc_hi = plsc.cumsum(hi) + inc_lo[L - 1]
        plsc.store_scatter(offb, [ch * 48 + iota], inc_lo - lo)
        plsc.store_scatter(offb, [ch * 48 + L + iota], inc_hi - hi)
        plsc.store_scatter(offb, [ch * 48 + 32 + iota],
                           jnp.full((L,), inc_hi[L - 1], jnp.float32))

    zi = jnp.zeros((L,), jnp.int32)
    for t in range((rs_v.shape[0] - PART_CHUNK) // L):
        rs_v[pl.ds(PART_CHUNK + t * L, L)] = zi
    pltpu.sync_copy(rs_hbm.at[pl.ds(base, PART_CHUNK + 16)],
                    rs_v.at[pl.ds(0, PART_CHUNK + 16)])
    pltpu.sync_copy(vflat.at[pl.ds(base * 3, PART_CHUNK * 3)], v_v)
    descs = []
    for r in range(rs_v.shape[0] // GB):
        isl = rs_v.at[pl.ds(r * GB, GB)]
        for src, buf in ((lx, gx), (ly, gy), (lz, gz), (la, ga)):
            descs.append(pltpu.async_copy(
                src.at[isl], buf.at[pl.ds(r * GB, GB)], sem))
    for dsc in descs:
        dsc.wait()

    mval = jnp.int32(M)

    def step(j, _):
        off = j * L
        pvec = iota + off
        rs_a = rs_v[pl.ds(off, L)]
        rs_b = plsc.load_gather(rs_v, [pvec + 1])
        ta_ = rs_a // epw
        tb_ = rs_b // epw
        svals = []
        for ch, buf in enumerate((ga, gx, gy, gz)):
            pa_ = buf[pl.ds(off, L)]
            pb_ = plsc.load_gather(buf, [pvec + 1])
            och = (3, 0, 1, 2)[ch]
            pa_ = jnp.where(rs_a == mval, 0.0, pa_)
            pb_ = jnp.where(rs_b == mval, 0.0, pb_)
            oa = plsc.load_gather(offb, [och * 48 + ta_])
            ob = plsc.load_gather(offb, [och * 48 + tb_])
            svals.append((pb_ - pa_) + (ob - oa))
        s0, s1x, s1y, s1z = svals
        pvec3 = pvec * 3
        for ch, s1 in enumerate((s1x, s1y, s1z)):
            vch = plsc.load_gather(v_v, [pvec3 + ch])
            plsc.store_scatter(o_v, [pvec3 + ch],
                               vch - visc * (vch * s0 - s1))
        return 0

    lax.fori_loop(0, PART_CHUNK // L, step, 0)

    @pl.when(w < n_chunks)
    def _():
        pltpu.sync_copy(o_v, oflat.at[pl.ds(base * 3, PART_CHUNK * 3)])


def kernel(velocities, masses, densities, neighbors_index,
           neighbors_row_splits, neighbors_distance):
    Np = velocities.shape[0]
    M = neighbors_index.shape[0]
    f32 = jnp.float32
    npad = NS * NPP - Np

    vflat = velocities.reshape(-1)
    vflat_pad = jnp.concatenate([vflat, jnp.zeros((npad * 3,), f32)])
    m_pad = jnp.concatenate([masses, jnp.ones((npad,), f32)])
    rho_pad = jnp.concatenate([densities, jnp.ones((npad,), f32)])

    mesh = plsc.VectorSubcoreMesh(core_axis_name="c", subcore_axis_name="s")
    lex_t = jax.ShapeDtypeStruct((M + 8,), f32)

    edge_kernel = pl.kernel(
        functools.partial(_edge_prefix_body, M, Np),
        out_type=(lex_t, lex_t, lex_t, lex_t,
                  jax.ShapeDtypeStruct((NW * L,), f32)),
        mesh=mesh,
        compiler_params=pltpu.CompilerParams(needs_layout_passes=False),
        scratch_types=(
            [pltpu.VMEM((NPP * 3,), f32)]
            + [pltpu.VMEM((NPP,), f32) for _ in range(6)]
            + [pltpu.VMEM_SHARED((NS * NPP,), f32) for _ in range(4)]
            + [pltpu.VMEM((2048,), jnp.int32), pltpu.VMEM((2048,), f32)]
            + [pltpu.VMEM((2048,), f32) for _ in range(4)]
            + [pltpu.VMEM((2048,), jnp.int32), pltpu.VMEM((2048,), f32)]
            + [pltpu.VMEM((2048,), f32) for _ in range(4)]
            + [pltpu.VMEM((EDGE_CHUNK,), f32) for _ in range(4)]
            + [pltpu.VMEM((L,), f32),
               pltpu.SemaphoreType.DMA, pltpu.SemaphoreType.DMA]
        ),
    )
    lx, ly, lz, la, tot = edge_kernel(vflat_pad, m_pad, rho_pad,
                                      neighbors_index, neighbors_distance)

    pad = jnp.full((63,), M, dtype=neighbors_row_splits.dtype)
    rs_pad = jnp.concatenate([neighbors_row_splits, pad])

    combine_kernel = pl.kernel(
        functools.partial(_combine_body, M, Np),
        out_type=jax.ShapeDtypeStruct((Np * 3,), f32),
        mesh=mesh,
        compiler_params=pltpu.CompilerParams(needs_layout_passes=False),
        scratch_types=(
            [pltpu.VMEM((2048,), jnp.int32)]
            + [pltpu.VMEM((2048,), f32) for _ in range(4)]
            + [pltpu.VMEM((PART_CHUNK * 3,), f32)]
            + [pltpu.VMEM((PART_CHUNK * 3,), f32)]
            + [pltpu.VMEM((NW * L,), f32), pltpu.VMEM((192,), f32),
               pltpu.SemaphoreType.DMA]
        ),
    )
    oflat = combine_kernel(vflat, rs_pad, lx, ly, lz, la, tot)
    return oflat.reshape(Np, 3)


# speedup vs baseline: 1.2036x; 1.1670x over previous
import functools

import jax
import jax.numpy as jnp
from jax import lax
from jax.experimental import pallas as pl
from jax.experimental.pallas import tpu as pltpu
from jax.experimental.pallas import tpu_sc as plsc

RADIUS = 0.1
VISCOSITY = 0.01

NC = 2
NS = 16
NW = NC * NS
L = 16

EDGE_CHUNK = 2000
PART_CHUNK = 2000
GB = 128
NPP = 3200


def _edge_prefix_body(M, Np, vflat, m_pad, rho_pad, nbr_idx, nbr_dist,
                      lx, ly, lz, la, tot,
                      vbuf, mbuf, rbuf, cx, cy, cz, ca,
                      spx, spy, spz, spa,
                      idx_a, dist_a, bxa, bya, bza, baa,
                      idx_b, dist_b, bxb, byb, bzb, bab,
                      px, py, pz, pa, tbuf, sem_a, sem_b):
    bufs_a = (bxa, bya, bza, baa)
    bufs_b = (bxb, byb, bzb, bab)
    prefs = (px, py, pz, pa)
    sps = (spx, spy, spz, spa)
    s = lax.axis_index("s")
    w = lax.axis_index("c") * NS + s
    iota = lax.iota(jnp.int32, L)
    epw = M // NW
    n_chunks = epw // EDGE_CHUNK
    inv_r2 = jnp.float32(1.0 / (RADIUS * RADIUS))
    zeros_i = jnp.zeros((L,), jnp.int32)

    p0 = s * NPP
    pltpu.sync_copy(vflat.at[pl.ds(p0 * 3, NPP * 3)], vbuf)
    pltpu.sync_copy(m_pad.at[pl.ds(p0, NPP)], mbuf)
    pltpu.sync_copy(rho_pad.at[pl.ds(p0, NPP)], rbuf)

    def bstep(j, _):
        off = j * L
        pvec3 = (iota + off) * 3
        cx[pl.ds(off, L)] = plsc.load_gather(vbuf, [pvec3])
        cy[pl.ds(off, L)] = plsc.load_gather(vbuf, [pvec3 + 1])
        cz[pl.ds(off, L)] = plsc.load_gather(vbuf, [pvec3 + 2])
        ca[pl.ds(off, L)] = mbuf[pl.ds(off, L)] / rbuf[pl.ds(off, L)]
        return 0

    lax.fori_loop(0, NPP // L, bstep, 0)
    for c, sp in zip((cx, cy, cz, ca), sps):
        pltpu.sync_copy(c, sp.at[pl.ds(p0, NPP)])
    plsc.subcore_barrier()

    def fire(c, idx_v, dist_v, bufs, sem):
        base = w * epw + c * EDGE_CHUNK
        pltpu.sync_copy(nbr_idx.at[pl.ds(base, EDGE_CHUNK)],
                        idx_v.at[pl.ds(0, EDGE_CHUNK)])
        pltpu.sync_copy(nbr_dist.at[pl.ds(base, EDGE_CHUNK)],
                        dist_v.at[pl.ds(0, EDGE_CHUNK)])
        for t in range((idx_v.shape[0] - EDGE_CHUNK) // L):
            idx_v[pl.ds(EDGE_CHUNK + t * L, L)] = zeros_i
        for r in range(idx_v.shape[0] // GB):
            isl = idx_v.at[pl.ds(r * GB, GB)]
            for sp, buf in zip(sps, bufs):
                pltpu.async_copy(sp.at[isl], buf.at[pl.ds(r * GB, GB)], sem)

    def drain(bufs, sem):
        for buf in bufs:
            pltpu.make_async_copy(nbr_dist.at[pl.ds(0, buf.shape[0])], buf,
                                  sem).wait()

    def compute_out(c, dist_v, bufs, carry):
        bx, by, bz, ba = bufs

        def step(j, carry):
            c0, c1, c2, c3 = carry
            off = j * L
            d = dist_v[pl.ds(off, L)]
            a = ba[pl.ds(off, L)]
            u = 1.0 - d * inv_r2
            wgt = jnp.clip(u * u * u, 0.0, 1.0) * a
            new_carry = []
            for ch, (x, cprev) in enumerate((
                    (wgt * bx[pl.ds(off, L)], c0),
                    (wgt * by[pl.ds(off, L)], c1),
                    (wgt * bz[pl.ds(off, L)], c2),
                    (wgt, c3))):
                inc = plsc.cumsum(x)
                prefs[ch][pl.ds(off, L)] = cprev + (inc - x)
                new_carry.append(cprev + inc[L - 1])
            return tuple(new_carry)

        carry = lax.fori_loop(0, EDGE_CHUNK // L, step, carry)
        base = w * epw + c * EDGE_CHUNK
        for pref, lex in zip(prefs, (lx, ly, lz, la)):
            pltpu.sync_copy(pref, lex.at[pl.ds(base, EDGE_CHUNK)])
        return carry

    fire(0, idx_a, dist_a, bufs_a, sem_a)

    def pair(k, carry):
        c0 = 2 * k
        drain(bufs_a, sem_a)
        fire(c0 + 1, idx_b, dist_b, bufs_b, sem_b)
        carry = compute_out(c0, dist_a, bufs_a, carry)
        drain(bufs_b, sem_b)
        fire(c0 + 2, idx_a, dist_a, bufs_a, sem_a)
        carry = compute_out(c0 + 1, dist_b, bufs_b, carry)
        return carry

    z = jnp.float32(0.0)
    carry = lax.fori_loop(0, (n_chunks - 1) // 2, pair, (z, z, z, z))
    drain(bufs_a, sem_a)
    c0, c1, c2, c3 = compute_out(n_chunks - 1, dist_a, bufs_a, carry)

    tvec = jnp.where(iota == 0, c0,
                     jnp.where(iota == 1, c1,
                               jnp.where(iota == 2, c2,
                                         jnp.where(iota == 3, c3, 0.0))))
    tbuf[...] = tvec
    pltpu.sync_copy(tbuf, tot.at[pl.ds(w * L, L)])


def _combine_body(M, Np, vflat, rs_hbm, lx, ly, lz, la, tot,
                  ox, oy, oz,
                  rs_v, gx, gy, gz, ga, v_v,
                  ox_v, oy_v, oz_v, t_v, offb, sem):
    w = lax.axis_index("c") * NS + lax.axis_index("s")
    iota = lax.iota(jnp.int32, L)
    epw = M // NW
    n_chunks = Np // PART_CHUNK
    visc = jnp.float32(VISCOSITY)

    base = lax.min(w, n_chunks - 1) * PART_CHUNK

    pltpu.sync_copy(tot, t_v)
    for ch in range(4):
        lo = plsc.load_gather(t_v, [iota * L + ch])
        hi = plsc.load_gather(t_v, [(iota + L) * L + ch])
        inc_lo = plsc.cumsum(lo)
        inc_hi = plsc.cumsum(hi) + inc_lo[L - 1]
        plsc.store_scatter(offb, [ch * 48 + iota], inc_lo - lo)
        plsc.store_scatter(offb, [ch * 48 + L + iota], inc_hi - hi)
        plsc.store_scatter(offb, [ch * 48 + 32 + iota],
                           jnp.full((L,), inc_hi[L - 1], jnp.float32))

    zi = jnp.zeros((L,), jnp.int32)
    for t in range((rs_v.shape[0] - PART_CHUNK) // L):
        rs_v[pl.ds(PART_CHUNK + t * L, L)] = zi
    pltpu.sync_copy(rs_hbm.at[pl.ds(base, PART_CHUNK + 16)],
                    rs_v.at[pl.ds(0, PART_CHUNK + 16)])
    pltpu.sync_copy(vflat.at[pl.ds(base * 3, PART_CHUNK * 3)], v_v)
    descs = []
    for r in range(rs_v.shape[0] // GB):
        isl = rs_v.at[pl.ds(r * GB, GB)]
        for src, buf in ((lx, gx), (ly, gy), (lz, gz), (la, ga)):
            descs.append(pltpu.async_copy(
                src.at[isl], buf.at[pl.ds(r * GB, GB)], sem))
    for dsc in descs:
        dsc.wait()

    mval = jnp.int32(M)

    def step(j, _):
        off = j * L
        pvec = iota + off
        rs_a = rs_v[pl.ds(off, L)]
        rs_b = plsc.load_gather(rs_v, [pvec + 1])
        ta_ = rs_a // epw
        tb_ = rs_b // epw
        svals = []
        for ch, buf in enumerate((ga, gx, gy, gz)):
            pa_ = buf[pl.ds(off, L)]
            pb_ = plsc.load_gather(buf, [pvec + 1])
            och = (3, 0, 1, 2)[ch]
            pa_ = jnp.where(rs_a == mval, 0.0, pa_)
            pb_ = jnp.where(rs_b == mval, 0.0, pb_)
            oa = plsc.load_gather(offb, [och * 48 + ta_])
            ob = plsc.load_gather(offb, [och * 48 + tb_])
            svals.append((pb_ - pa_) + (ob - oa))
        s0, s1x, s1y, s1z = svals
        pvec3 = pvec * 3
        for ch, (obuf, s1) in enumerate(((ox_v, s1x), (oy_v, s1y),
                                         (oz_v, s1z))):
            vch = plsc.load_gather(v_v, [pvec3 + ch])
            obuf[pl.ds(off, L)] = vch - visc * (vch * s0 - s1)
        return 0

    lax.fori_loop(0, PART_CHUNK // L, step, 0)

    @pl.when(w < n_chunks)
    def _():
        pltpu.sync_copy(ox_v, ox.at[pl.ds(base, PART_CHUNK)])
        pltpu.sync_copy(oy_v, oy.at[pl.ds(base, PART_CHUNK)])
        pltpu.sync_copy(oz_v, oz.at[pl.ds(base, PART_CHUNK)])


def kernel(velocities, masses, densities, neighbors_index,
           neighbors_row_splits, neighbors_distance):
    Np = velocities.shape[0]
    M = neighbors_index.shape[0]
    f32 = jnp.float32
    npad = NS * NPP - Np

    vflat = velocities.reshape(-1)
    vflat_pad = jnp.concatenate([vflat, jnp.zeros((npad * 3,), f32)])
    m_pad = jnp.concatenate([masses, jnp.ones((npad,), f32)])
    rho_pad = jnp.concatenate([densities, jnp.ones((npad,), f32)])

    mesh = plsc.VectorSubcoreMesh(core_axis_name="c", subcore_axis_name="s")
    lex_t = jax.ShapeDtypeStruct((M + 8,), f32)

    edge_kernel = pl.kernel(
        functools.partial(_edge_prefix_body, M, Np),
        out_type=(lex_t, lex_t, lex_t, lex_t,
                  jax.ShapeDtypeStruct((NW * L,), f32)),
        mesh=mesh,
        compiler_params=pltpu.CompilerParams(needs_layout_passes=False),
        scratch_types=(
            [pltpu.VMEM((NPP * 3,), f32)]
            + [pltpu.VMEM((NPP,), f32) for _ in range(6)]
            + [pltpu.VMEM_SHARED((NS * NPP,), f32) for _ in range(4)]
            + [pltpu.VMEM((2048,), jnp.int32), pltpu.VMEM((2048,), f32)]
            + [pltpu.VMEM((2048,), f32) for _ in range(4)]
            + [pltpu.VMEM((2048,), jnp.int32), pltpu.VMEM((2048,), f32)]
            + [pltpu.VMEM((2048,), f32) for _ in range(4)]
            + [pltpu.VMEM((EDGE_CHUNK,), f32) for _ in range(4)]
            + [pltpu.VMEM((L,), f32),
               pltpu.SemaphoreType.DMA, pltpu.SemaphoreType.DMA]
        ),
    )
    lx, ly, lz, la, tot = edge_kernel(vflat_pad, m_pad, rho_pad,
                                      neighbors_index, neighbors_distance)

    pad = jnp.full((63,), M, dtype=neighbors_row_splits.dtype)
    rs_pad = jnp.concatenate([neighbors_row_splits, pad])

    ocol = jax.ShapeDtypeStruct((Np,), f32)
    combine_kernel = pl.kernel(
        functools.partial(_combine_body, M, Np),
        out_type=(ocol, ocol, ocol),
        mesh=mesh,
        compiler_params=pltpu.CompilerParams(needs_layout_passes=False),
        scratch_types=(
            [pltpu.VMEM((2048,), jnp.int32)]
            + [pltpu.VMEM((2048,), f32) for _ in range(4)]
            + [pltpu.VMEM((PART_CHUNK * 3,), f32)]
            + [pltpu.VMEM((PART_CHUNK,), f32) for _ in range(3)]
            + [pltpu.VMEM((NW * L,), f32), pltpu.VMEM((192,), f32),
               pltpu.SemaphoreType.DMA]
        ),
    )
    ox, oy, oz = combine_kernel(vflat, rs_pad, lx, ly, lz, la, tot)
    return jnp.stack([ox, oy, oz], axis=1)


# speedup vs baseline: 1.2053x; 1.0014x over previous
import functools

import jax
import jax.numpy as jnp
from jax import lax
from jax.experimental import pallas as pl
from jax.experimental.pallas import tpu as pltpu
from jax.experimental.pallas import tpu_sc as plsc

RADIUS = 0.1
VISCOSITY = 0.01

NC = 2
NS = 16
NW = NC * NS
L = 16

EDGE_CHUNK = 2000
PART_CHUNK = 2000
GB = 128
NPP = 3200


def _edge_prefix_body(M, Np, vflat, m_pad, rho_pad, nbr_idx, nbr_dist,
                      lx, ly, lz, la, tot,
                      vbuf, mbuf, rbuf, cx, cy, cz, ca,
                      spx, spy, spz, spa,
                      idx_a, dist_a, bxa, bya, bza, baa,
                      idx_b, dist_b, bxb, byb, bzb, bab,
                      px, py, pz, pa, tbuf, sem_a, sem_b):
    bufs_a = (bxa, bya, bza, baa)
    bufs_b = (bxb, byb, bzb, bab)
    prefs = (px, py, pz, pa)
    sps = (spx, spy, spz, spa)
    s = lax.axis_index("s")
    w = lax.axis_index("c") * NS + s
    iota = lax.iota(jnp.int32, L)
    epw = M // NW
    n_chunks = epw // EDGE_CHUNK
    inv_r2 = jnp.float32(1.0 / (RADIUS * RADIUS))
    zeros_i = jnp.zeros((L,), jnp.int32)

    p0 = s * NPP
    pltpu.sync_copy(vflat.at[pl.ds(p0 * 3, NPP * 3)], vbuf)
    pltpu.sync_copy(m_pad.at[pl.ds(p0, NPP)], mbuf)
    pltpu.sync_copy(rho_pad.at[pl.ds(p0, NPP)], rbuf)

    def bstep(j, _):
        off = j * L
        pvec3 = (iota + off) * 3
        cx[pl.ds(off, L)] = plsc.load_gather(vbuf, [pvec3])
        cy[pl.ds(off, L)] = plsc.load_gather(vbuf, [pvec3 + 1])
        cz[pl.ds(off, L)] = plsc.load_gather(vbuf, [pvec3 + 2])
        ca[pl.ds(off, L)] = mbuf[pl.ds(off, L)] / rbuf[pl.ds(off, L)]
        return 0

    lax.fori_loop(0, NPP // L, bstep, 0)
    for c, sp in zip((cx, cy, cz, ca), sps):
        pltpu.sync_copy(c, sp.at[pl.ds(p0, NPP)])
    plsc.subcore_barrier()

    def fire(c, idx_v, dist_v, bufs, sem):
        base = w * epw + c * EDGE_CHUNK
        pltpu.sync_copy(nbr_idx.at[pl.ds(base, EDGE_CHUNK)],
                        idx_v.at[pl.ds(0, EDGE_CHUNK)])
        pltpu.sync_copy(nbr_dist.at[pl.ds(base, EDGE_CHUNK)],
                        dist_v.at[pl.ds(0, EDGE_CHUNK)])
        for t in range((idx_v.shape[0] - EDGE_CHUNK) // L):
            idx_v[pl.ds(EDGE_CHUNK + t * L, L)] = zeros_i
        for r in range(idx_v.shape[0] // GB):
            isl = idx_v.at[pl.ds(r * GB, GB)]
            for sp, buf in zip(sps, bufs):
                pltpu.async_copy(sp.at[isl], buf.at[pl.ds(r * GB, GB)], sem)

    def drain(bufs, sem):
        for buf in bufs:
            pltpu.make_async_copy(nbr_dist.at[pl.ds(0, buf.shape[0])], buf,
                                  sem).wait()

    def compute_out(c, dist_v, bufs, carry):
        bx, by, bz, ba = bufs

        def halfstep(off, carry):
            c0, c1, c2, c3 = carry
            d = dist_v[pl.ds(off, L)]
            a = ba[pl.ds(off, L)]
            u = 1.0 - d * inv_r2
            wgt = jnp.clip(u * u * u, 0.0, 1.0) * a
            incs = []
            xs = (wgt * bx[pl.ds(off, L)], wgt * by[pl.ds(off, L)],
                  wgt * bz[pl.ds(off, L)], wgt)
            for x in xs:
                incs.append(plsc.cumsum(x))
            new_carry = []
            for ch, (x, inc, cprev) in enumerate(zip(xs, incs,
                                                     (c0, c1, c2, c3))):
                prefs[ch][pl.ds(off, L)] = cprev + (inc - x)
                new_carry.append(cprev + inc[L - 1])
            return tuple(new_carry)

        def step2(j, carry):
            carry = halfstep(j * (2 * L), carry)
            return halfstep(j * (2 * L) + L, carry)

        carry = lax.fori_loop(0, EDGE_CHUNK // (2 * L), step2, carry)
        if EDGE_CHUNK % (2 * L):
            carry = halfstep(EDGE_CHUNK - L, carry)
        base = w * epw + c * EDGE_CHUNK
        for pref, lex in zip(prefs, (lx, ly, lz, la)):
            pltpu.sync_copy(pref, lex.at[pl.ds(base, EDGE_CHUNK)])
        return carry

    fire(0, idx_a, dist_a, bufs_a, sem_a)

    def pair(k, carry):
        c0 = 2 * k
        drain(bufs_a, sem_a)
        fire(c0 + 1, idx_b, dist_b, bufs_b, sem_b)
        carry = compute_out(c0, dist_a, bufs_a, carry)
        drain(bufs_b, sem_b)
        fire(c0 + 2, idx_a, dist_a, bufs_a, sem_a)
        carry = compute_out(c0 + 1, dist_b, bufs_b, carry)
        return carry

    z = jnp.float32(0.0)
    carry = lax.fori_loop(0, (n_chunks - 1) // 2, pair, (z, z, z, z))
    drain(bufs_a, sem_a)
    c0, c1, c2, c3 = compute_out(n_chunks - 1, dist_a, bufs_a, carry)

    tvec = jnp.where(iota == 0, c0,
                     jnp.where(iota == 1, c1,
                               jnp.where(iota == 2, c2,
                                         jnp.where(iota == 3, c3, 0.0))))
    tbuf[...] = tvec
    pltpu.sync_copy(tbuf, tot.at[pl.ds(w * L, L)])


def _combine_body(M, Np, vflat, rs_hbm, lx, ly, lz, la, tot,
                  ox, oy, oz,
                  rs_v, gx, gy, gz, ga, v_v,
                  ox_v, oy_v, oz_v, t_v, offb, sem):
    w = lax.axis_index("c") * NS + lax.axis_index("s")
    iota = lax.iota(jnp.int32, L)
    epw = M // NW
    n_chunks = Np // PART_CHUNK
    visc = jnp.float32(VISCOSITY)

    base = lax.min(w, n_chunks - 1) * PART_CHUNK

    pltpu.sync_copy(tot, t_v)
    for ch in range(4):
        lo = plsc.load_gather(t_v, [iota * L + ch])
        hi = plsc.load_gather(t_v, [(iota + L) * L + ch])
        inc_lo = plsc.cumsum(lo)
        inc_hi = plsc.cumsum(hi) + inc_lo[L - 1]
        plsc.store_scatter(offb, [ch * 48 + iota], inc_lo - lo)
        plsc.store_scatter(offb, [ch * 48 + L + iota], inc_hi - hi)
        plsc.store_scatter(offb, [ch * 48 + 32 + iota],
                           jnp.full((L,), inc_hi[L - 1], jnp.float32))

    zi = jnp.zeros((L,), jnp.int32)
    for t in range((rs_v.shape[0] - PART_CHUNK) // L):
        rs_v[pl.ds(PART_CHUNK + t * L, L)] = zi
    pltpu.sync_copy(rs_hbm.at[pl.ds(base, PART_CHUNK + 16)],
                    rs_v.at[pl.ds(0, PART_CHUNK + 16)])
    pltpu.sync_copy(vflat.at[pl.ds(base * 3, PART_CHUNK * 3)], v_v)
    descs = []
    for r in range(rs_v.shape[0] // GB):
        isl = rs_v.at[pl.ds(r * GB, GB)]
        for src, buf in ((lx, gx), (ly, gy), (lz, gz), (la, ga)):
            descs.append(pltpu.async_copy(
                src.at[isl], buf.at[pl.ds(r * GB, GB)], sem))
    for dsc in descs:
        dsc.wait()

    mval = jnp.int32(M)

    def step(j, _):
        off = j * L
        pvec = iota + off
        rs_a = rs_v[pl.ds(off, L)]
        rs_b = plsc.load_gather(rs_v, [pvec + 1])
        ta_ = rs_a // epw
        tb_ = rs_b // epw
        svals = []
        for ch, buf in enumerate((ga, gx, gy, gz)):
            pa_ = buf[pl.ds(off, L)]
            pb_ = plsc.load_gather(buf, [pvec + 1])
            och = (3, 0, 1, 2)[ch]
            pa_ = jnp.where(rs_a == mval, 0.0, pa_)
            pb_ = jnp.where(rs_b == mval, 0.0, pb_)
            oa = plsc.load_gather(offb, [och * 48 + ta_])
            ob = plsc.load_gather(offb, [och * 48 + tb_])
            svals.append((pb_ - pa_) + (ob - oa))
        s0, s1x, s1y, s1z = svals
        pvec3 = pvec * 3
        for ch, (obuf, s1) in enumerate(((ox_v, s1x), (oy_v, s1y),
                                         (oz_v, s1z))):
            vch = plsc.load_gather(v_v, [pvec3 + ch])
            obuf[pl.ds(off, L)] = vch - visc * (vch * s0 - s1)
        return 0

    lax.fori_loop(0, PART_CHUNK // L, step, 0)

    @pl.when(w < n_chunks)
    def _():
        pltpu.sync_copy(ox_v, ox.at[pl.ds(base, PART_CHUNK)])
        pltpu.sync_copy(oy_v, oy.at[pl.ds(base, PART_CHUNK)])
        pltpu.sync_copy(oz_v, oz.at[pl.ds(base, PART_CHUNK)])


def kernel(velocities, masses, densities, neighbors_index,
           neighbors_row_splits, neighbors_distance):
    Np = velocities.shape[0]
    M = neighbors_index.shape[0]
    f32 = jnp.float32
    npad = NS * NPP - Np

    vflat = velocities.reshape(-1)
    vflat_pad = jnp.concatenate([vflat, jnp.zeros((npad * 3,), f32)])
    m_pad = jnp.concatenate([masses, jnp.ones((npad,), f32)])
    rho_pad = jnp.concatenate([densities, jnp.ones((npad,), f32)])

    mesh = plsc.VectorSubcoreMesh(core_axis_name="c", subcore_axis_name="s")
    lex_t = jax.ShapeDtypeStruct((M + 8,), f32)

    edge_kernel = pl.kernel(
        functools.partial(_edge_prefix_body, M, Np),
        out_type=(lex_t, lex_t, lex_t, lex_t,
                  jax.ShapeDtypeStruct((NW * L,), f32)),
        mesh=mesh,
        compiler_params=pltpu.CompilerParams(needs_layout_passes=False),
        scratch_types=(
            [pltpu.VMEM((NPP * 3,), f32)]
            + [pltpu.VMEM((NPP,), f32) for _ in range(6)]
            + [pltpu.VMEM_SHARED((NS * NPP,), f32) for _ in range(4)]
            + [pltpu.VMEM((2048,), jnp.int32), pltpu.VMEM((2048,), f32)]
            + [pltpu.VMEM((2048,), f32) for _ in range(4)]
            + [pltpu.VMEM((2048,), jnp.int32), pltpu.VMEM((2048,), f32)]
            + [pltpu.VMEM((2048,), f32) for _ in range(4)]
            + [pltpu.VMEM((EDGE_CHUNK,), f32) for _ in range(4)]
            + [pltpu.VMEM((L,), f32),
               pltpu.SemaphoreType.DMA, pltpu.SemaphoreType.DMA]
        ),
    )
    lx, ly, lz, la, tot = edge_kernel(vflat_pad, m_pad, rho_pad,
                                      neighbors_index, neighbors_distance)

    pad = jnp.full((63,), M, dtype=neighbors_row_splits.dtype)
    rs_pad = jnp.concatenate([neighbors_row_splits, pad])

    ocol = jax.ShapeDtypeStruct((Np,), f32)
    combine_kernel = pl.kernel(
        functools.partial(_combine_body, M, Np),
        out_type=(ocol, ocol, ocol),
        mesh=mesh,
        compiler_params=pltpu.CompilerParams(needs_layout_passes=False),
        scratch_types=(
            [pltpu.VMEM((2048,), jnp.int32)]
            + [pltpu.VMEM((2048,), f32) for _ in range(4)]
            + [pltpu.VMEM((PART_CHUNK * 3,), f32)]
            + [pltpu.VMEM((PART_CHUNK,), f32) for _ in range(3)]
            + [pltpu.VMEM((NW * L,), f32), pltpu.VMEM((192,), f32),
               pltpu.SemaphoreType.DMA]
        ),
    )
    ox, oy, oz = combine_kernel(vflat, rs_pad, lx, ly, lz, la, tot)
    return jnp.stack([ox, oy, oz], axis=1)
